# baseline (speedup 1.0000x reference)
import jax
import jax.numpy as jnp
from jax import lax
from jax.experimental import pallas as pl
from jax.experimental.pallas import tpu as pltpu

RATE = 1.0
B, C_IN, H, W = 8, 192, 224, 224
C_OUT = 192
K = int(C_OUT * RATE)
SPATIAL = H * W
ROWS = B * C_IN
LANES = 128
COL_BLOCK = 1792
NSTEPS = SPATIAL // COL_BLOCK


def _reduce_kernel(x_ref, acc_ref):
    j = pl.program_id(0)
    a = jnp.abs(x_ref[...])
    a3 = a.reshape(ROWS, COL_BLOCK // LANES, LANES)
    part = jnp.sum(a3, axis=1)

    @pl.when(j == 0)
    def _init():
        acc_ref[...] = part

    @pl.when(j > 0)
    def _acc():
        acc_ref[...] = acc_ref[...] + part


def _gate_mask_kernel(sp_ref, w_ref, b_ref, t_ref):
    s = jnp.sum(sp_ref[...], axis=2) * (1.0 / SPATIAL)
    g = lax.dot_general(s, w_ref[...], (((1,), (1,)), ((), ())),
                        preferred_element_type=jnp.float32)
    g = jnp.maximum(g + b_ref[...], 0.0)
    ge = g[:, :, None]
    gm = g[:, None, :]
    e_idx = lax.broadcasted_iota(jnp.int32, (B, C_OUT, C_OUT), 1)
    m_idx = lax.broadcasted_iota(jnp.int32, (B, C_OUT, C_OUT), 2)
    smaller = (gm < ge) | ((gm == ge) & (m_idx < e_idx))
    rank = jnp.sum(smaller.astype(jnp.int32), axis=2)
    t = jnp.where(rank >= K, g, 0.0)
    t_sum = jnp.sum(t, axis=1, keepdims=True)
    t_ref[...] = t / t_sum * C_OUT


@jax.jit
def kernel(x, gate_w, gate_b):
    x2 = x.reshape(ROWS, SPATIAL)
    w_main = gate_w[:, :C_IN]
    b_eff = (gate_b + RATE * gate_w[:, C_IN]).reshape(1, C_OUT)

    s_part = pl.pallas_call(
        _reduce_kernel,
        grid=(NSTEPS,),
        in_specs=[pl.BlockSpec((ROWS, COL_BLOCK), lambda j: (0, j))],
        out_specs=pl.BlockSpec((ROWS, LANES), lambda j: (0, 0)),
        out_shape=jax.ShapeDtypeStruct((ROWS, LANES), jnp.float32),
    )(x2)

    t = pl.pallas_call(
        _gate_mask_kernel,
        out_shape=jax.ShapeDtypeStruct((B, C_OUT), jnp.float32),
    )(s_part.reshape(B, C_IN, LANES), w_main, b_eff)
    return t[:, :, None, None]


# speedup vs baseline: 1.1539x; 1.1539x over previous
import jax
import jax.numpy as jnp
from jax import lax
from jax.experimental import pallas as pl
from jax.experimental.pallas import tpu as pltpu

RATE = 1.0
B, C_IN, H, W = 8, 192, 224, 224
C_OUT = 192
K = int(C_OUT * RATE)
SPATIAL = H * W
ROWS = B * C_IN
LANES = 128
COL_BLOCK = 1792
NSTEPS = SPATIAL // COL_BLOCK


def _reduce_kernel(x_ref, acc_ref):
    j = pl.program_id(0)
    a = jnp.abs(x_ref[...])
    part = a[:, 0:LANES]
    for c in range(1, COL_BLOCK // LANES):
        part = part + a[:, c * LANES:(c + 1) * LANES]

    @pl.when(j == 0)
    def _init():
        acc_ref[...] = part

    @pl.when(j > 0)
    def _acc():
        acc_ref[...] = acc_ref[...] + part


def _gate_mask_kernel(sp_ref, w_ref, b_ref, t_ref):
    s = jnp.sum(sp_ref[...], axis=2) * (1.0 / SPATIAL)
    g = lax.dot_general(s, w_ref[...], (((1,), (1,)), ((), ())),
                        preferred_element_type=jnp.float32)
    g = jnp.maximum(g + b_ref[...], 0.0)
    ge = g[:, :, None]
    gm = g[:, None, :]
    e_idx = lax.broadcasted_iota(jnp.int32, (B, C_OUT, C_OUT), 1)
    m_idx = lax.broadcasted_iota(jnp.int32, (B, C_OUT, C_OUT), 2)
    smaller = (gm < ge) | ((gm == ge) & (m_idx < e_idx))
    rank = jnp.sum(smaller.astype(jnp.int32), axis=2)
    t = jnp.where(rank >= K, g, 0.0)
    t_sum = jnp.sum(t, axis=1, keepdims=True)
    t_ref[...] = t / t_sum * C_OUT


@jax.jit
def kernel(x, gate_w, gate_b):
    x2 = x.reshape(ROWS, SPATIAL)
    w_main = gate_w[:, :C_IN]
    b_eff = (gate_b + RATE * gate_w[:, C_IN]).reshape(1, C_OUT)

    s_part = pl.pallas_call(
        _reduce_kernel,
        grid=(NSTEPS,),
        in_specs=[pl.BlockSpec((ROWS, COL_BLOCK), lambda j: (0, j))],
        out_specs=pl.BlockSpec((ROWS, LANES), lambda j: (0, 0)),
        out_shape=jax.ShapeDtypeStruct((ROWS, LANES), jnp.float32),
    )(x2)

    t = pl.pallas_call(
        _gate_mask_kernel,
        out_shape=jax.ShapeDtypeStruct((B, C_OUT), jnp.float32),
    )(s_part.reshape(B, C_IN, LANES), w_main, b_eff)
    return t[:, :, None, None]


# speedup vs baseline: 4.3966x; 3.8104x over previous
import jax
import jax.numpy as jnp
from jax import lax
from jax.experimental import pallas as pl
from jax.experimental.pallas import tpu as pltpu

RATE = 1.0
B, C_IN, H, W = 8, 192, 224, 224
C_OUT = 192
K = int(C_OUT * RATE)
SPATIAL = H * W
CB = 32
NCB = C_IN // CB


def _reduce_kernel(x_ref, out_ref):
    a = jnp.abs(x_ref[...])
    out_ref[...] = jnp.sum(a, axis=2)


def _gate_mask_kernel(sp_ref, w_ref, b_ref, t_ref):
    s = jnp.sum(sp_ref[...], axis=2) * (1.0 / SPATIAL)
    g = lax.dot_general(s, w_ref[...], (((1,), (1,)), ((), ())),
                        preferred_element_type=jnp.float32)
    g = jnp.maximum(g + b_ref[...], 0.0)
    ge = g[:, :, None]
    gm = g[:, None, :]
    e_idx = lax.broadcasted_iota(jnp.int32, (B, C_OUT, C_OUT), 1)
    m_idx = lax.broadcasted_iota(jnp.int32, (B, C_OUT, C_OUT), 2)
    smaller = (gm < ge) | ((gm == ge) & (m_idx < e_idx))
    rank = jnp.sum(smaller.astype(jnp.int32), axis=2)
    t = jnp.where(rank >= K, g, 0.0)
    t_sum = jnp.sum(t, axis=1, keepdims=True)
    t_ref[...] = t / t_sum * C_OUT


@jax.jit
def kernel(x, gate_w, gate_b):
    w_main = gate_w[:, :C_IN]
    b_eff = (gate_b + RATE * gate_w[:, C_IN]).reshape(1, C_OUT)

    s_part = pl.pallas_call(
        _reduce_kernel,
        grid=(B, NCB),
        in_specs=[pl.BlockSpec((1, CB, H, W), lambda b, c: (b, c, 0, 0))],
        out_specs=pl.BlockSpec((1, CB, W), lambda b, c: (b, c, 0)),
        out_shape=jax.ShapeDtypeStruct((B, C_IN, W), jnp.float32),
    )(x)

    t = pl.pallas_call(
        _gate_mask_kernel,
        out_shape=jax.ShapeDtypeStruct((B, C_OUT), jnp.float32),
    )(s_part, w_main, b_eff)
    return t[:, :, None, None]
